# baseline (speedup 1.0000x reference)
import functools

import jax
import jax.numpy as jnp
from jax.experimental import pallas as pl

NUM_EXPERTS = 8
TOP_K = 4
ALPHA = 1.702
LIMIT = 7.0
CHUNK = 512


def _moe_kernel(x_ref, wr_ref, rb_ref, wg_ref, wu_ref, bg_ref, bu_ref,
                wd_ref, bd_ref, out_ref, fs_ref, ti_ref):
    e = pl.program_id(0)
    T = x_ref.shape[0]

    @pl.when(e == 0)
    def _router():
        x = x_ref[...]
        logits = jax.lax.dot_general(
            x.astype(jnp.bfloat16), wr_ref[...].astype(jnp.bfloat16),
            (((1,), (0,)), ((), ())),
            preferred_element_type=jnp.float32) + rb_ref[...]
        m = jnp.max(logits, axis=-1, keepdims=True)
        p = jnp.exp(logits - m)
        scores = p / jnp.sum(p, axis=-1, keepdims=True)

        iota = jax.lax.broadcasted_iota(jnp.int32, scores.shape, 1)
        masked = scores
        fs = jnp.zeros_like(scores)
        for k in range(TOP_K):
            mv = jnp.max(masked, axis=-1, keepdims=True)
            idx = jnp.min(jnp.where(masked == mv, iota, NUM_EXPERTS),
                          axis=-1, keepdims=True)
            ti_ref[:, k:k + 1] = idx
            sel = iota == idx
            fs = jnp.where(sel, masked, fs)
            masked = jnp.where(sel, -jnp.inf, masked)
        fs_ref[...] = fs

    for c in range(T // CHUNK):
        sl = slice(c * CHUNK, (c + 1) * CHUNK)
        e_col = jax.lax.broadcasted_iota(
            jnp.int32, (CHUNK, NUM_EXPERTS), 1) == e
        gate_w = jnp.sum(jnp.where(e_col, fs_ref[sl, :], 0.0), axis=-1,
                         keepdims=True)
        xb = x_ref[sl, :].astype(jnp.bfloat16)
        g = jax.lax.dot_general(
            xb, wg_ref[0], (((1,), (0,)), ((), ())),
            preferred_element_type=jnp.float32) + bg_ref[0]
        u = jax.lax.dot_general(
            xb, wu_ref[0], (((1,), (0,)), ((), ())),
            preferred_element_type=jnp.float32) + bu_ref[0]
        g = jnp.minimum(g, LIMIT)
        u = jnp.clip(u, -LIMIT, LIMIT)
        glu = g * jax.nn.sigmoid(g * ALPHA)
        act = ((u + 1.0) * glu).astype(jnp.bfloat16)
        o = jax.lax.dot_general(
            act, wd_ref[0], (((1,), (0,)), ((), ())),
            preferred_element_type=jnp.float32) + bd_ref[0]
        contrib = gate_w * o

        @pl.when(e == 0)
        def _init():
            out_ref[sl, :] = contrib

        @pl.when(e != 0)
        def _acc():
            out_ref[sl, :] += contrib


def kernel(hidden_states, router_weight, router_bias, gate_up_proj,
           gate_up_proj_bias, down_proj, down_proj_bias):
    B, S, H = hidden_states.shape
    T = B * S
    E, _, F2 = gate_up_proj.shape
    F = F2 // 2

    x = hidden_states.reshape(T, H)
    wr = router_weight.T
    rb = router_bias.reshape(1, E)
    wg = gate_up_proj[:, :, 0::2].astype(jnp.bfloat16)
    wu = gate_up_proj[:, :, 1::2].astype(jnp.bfloat16)
    bg = gate_up_proj_bias[:, 0::2].reshape(E, 1, F)
    bu = gate_up_proj_bias[:, 1::2].reshape(E, 1, F)
    wd = down_proj.astype(jnp.bfloat16)
    bd = down_proj_bias.reshape(E, 1, H)

    const = lambda *shape: pl.BlockSpec(shape, lambda e: (0,) * len(shape))
    per_e3 = lambda d1, d2: pl.BlockSpec((1, d1, d2), lambda e: (e, 0, 0))

    out, fs, ti = pl.pallas_call(
        _moe_kernel,
        grid=(E,),
        in_specs=[
            const(T, H),
            const(H, E),
            const(1, E),
            per_e3(H, F),
            per_e3(H, F),
            per_e3(1, F),
            per_e3(1, F),
            per_e3(F, H),
            per_e3(1, H),
        ],
        out_specs=[
            const(T, H),
            const(T, NUM_EXPERTS),
            const(T, TOP_K),
        ],
        out_shape=[
            jax.ShapeDtypeStruct((T, H), jnp.float32),
            jax.ShapeDtypeStruct((T, NUM_EXPERTS), jnp.float32),
            jax.ShapeDtypeStruct((T, TOP_K), jnp.int32),
        ],
    )(x, wr, rb, wg, wu, bg, bu, wd, bd)

    return out.reshape(B, S, H), fs, ti
